# baseline (speedup 1.0000x reference)
import functools

import jax
import jax.numpy as jnp
from jax import lax
from jax.experimental import pallas as pl
from jax.experimental.pallas import tpu as pltpu
from jax.experimental.pallas import tpu_sc as plsc

_B, _H, _W = 8, 512, 512
_P = _B * _H * _W
_NC, _NS, _L = 2, 16, 16
_NW = _NC * _NS
_PIX_PER_W = _P // _NW
_CHUNK = 4096
_NCHUNK = _PIX_PER_W // _CHUNK
_GB = 128
_NGB = _CHUNK // _GB


def _floor_parts(g):
    t = g.astype(jnp.int32)
    tf = t.astype(jnp.float32)
    f = jnp.where(tf > g, tf - 1.0, tf)
    return f.astype(jnp.int32), g - f


def _sc_body(ttop, tbot, gx_hbm, gy_hbm, out_hbm,
             gxv, gyv, idxv, ptv, pbv,
             wav, wbv, wcv, wdv, outv, tmpi, sem0, sem1):
    wid = lax.axis_index("s") * _NC + lax.axis_index("c")

    def compute_and_fire(c, q):
        base = wid * _PIX_PER_W + c * _CHUNK
        bbase = (base >> 18) << 18
        pltpu.sync_copy(gx_hbm.at[pl.ds(base, _CHUNK)], gxv)
        pltpu.sync_copy(gy_hbm.at[pl.ds(base, _CHUNK)], gyv)

        @plsc.parallel_loop(0, _CHUNK // _L, 1, unroll=8)
        def idx_body(i):
            off = i * _L
            gx = gxv[pl.ds(off, _L)]
            gy = gyv[pl.ds(off, _L)]
            x0, fxr = _floor_parts(gx)
            y0, fyr = _floor_parts(gy)
            exr = 1.0 - fxr
            eyr = 1.0 - fyr
            wa = exr * eyr
            wb = fxr * eyr
            wc = exr * fyr
            wd = fxr * fyr
            zero = jnp.zeros_like(wa)
            mx = gx < 0.0
            wa = jnp.where(mx, wa + wb, wa)
            wb = jnp.where(mx, zero, wb)
            wc = jnp.where(mx, wc + wd, wc)
            wd = jnp.where(mx, zero, wd)
            my = gy < 0.0
            wa = jnp.where(my, wa + wc, wa)
            wc = jnp.where(my, zero, wc)
            wb = jnp.where(my, wb + wd, wb)
            wd = jnp.where(my, zero, wd)
            x0c = jnp.minimum(jnp.maximum(x0, 0), _W - 1)
            y0c = jnp.minimum(jnp.maximum(y0, 0), _H - 1)
            gidx = bbase + (y0c << 9) + x0c
            idxv[q, pl.ds(off, _L)] = gidx
            wav[q, pl.ds(off, _L)] = wa
            wbv[q, pl.ds(off, _L)] = wb
            wcv[q, pl.ds(off, _L)] = wc
            wdv[q, pl.ds(off, _L)] = wd

        sem = sem0 if q == 0 else sem1

        def fire(j, _):
            sl = pl.ds(j * _GB, _GB)
            isl = idxv.at[q].at[sl]
            pltpu.async_copy(ttop.at[isl], ptv.at[q].at[sl], sem)
            pltpu.async_copy(tbot.at[isl], pbv.at[q].at[sl], sem)
            return 0

        lax.fori_loop(0, _NGB, fire, 0)

    def drain_and_blend(c, q):
        base = wid * _PIX_PER_W + c * _CHUNK
        sem = sem0 if q == 0 else sem1
        dummy = ttop.at[pl.ds(0, _CHUNK)]
        pltpu.make_async_copy(dummy, ptv.at[q], sem).wait()
        pltpu.make_async_copy(dummy, pbv.at[q], sem).wait()
        tmpf = tmpi.bitcast(jnp.float32)

        @plsc.parallel_loop(0, _CHUNK // _L, 1, unroll=8)
        def blend_body(i):
            off = i * _L
            sl = pl.ds(off, _L)
            pt = ptv[q, sl]
            pbt = pbv[q, sl]
            himask = jnp.full_like(pt, -65536)
            tmpi[0, sl] = pt << 16
            tmpi[1, sl] = pt & himask
            tmpi[2, sl] = pbt << 16
            tmpi[3, sl] = pbt & himask
            o = (wav[q, sl] * tmpf[0, sl] + wbv[q, sl] * tmpf[1, sl]
                 + wcv[q, sl] * tmpf[2, sl] + wdv[q, sl] * tmpf[3, sl])
            outv[sl] = o
        pltpu.sync_copy(outv, out_hbm.at[pl.ds(base, _CHUNK)])

    compute_and_fire(0, 0)

    def chunk_pair(m, _):
        c = 2 * m + 1
        compute_and_fire(c, 1)
        drain_and_blend(c - 1, 0)
        compute_and_fire(c + 1, 0)
        drain_and_blend(c, 1)
        return 0

    lax.fori_loop(0, (_NCHUNK - 2) // 2, chunk_pair, 0)
    compute_and_fire(_NCHUNK - 1, 1)
    drain_and_blend(_NCHUNK - 2, 0)
    drain_and_blend(_NCHUNK - 1, 1)


_sc_call = functools.partial(
    pl.kernel,
    out_type=jax.ShapeDtypeStruct((_P,), jnp.float32),
    mesh=plsc.VectorSubcoreMesh(core_axis_name="c", subcore_axis_name="s",
                                num_cores=_NC, num_subcores=_NS),
    scratch_types=[
        pltpu.VMEM((_CHUNK,), jnp.float32),
        pltpu.VMEM((_CHUNK,), jnp.float32),
        pltpu.VMEM((2, _CHUNK), jnp.int32),
        pltpu.VMEM((2, _CHUNK), jnp.int32),
        pltpu.VMEM((2, _CHUNK), jnp.int32),
        pltpu.VMEM((2, _CHUNK), jnp.float32),
        pltpu.VMEM((2, _CHUNK), jnp.float32),
        pltpu.VMEM((2, _CHUNK), jnp.float32),
        pltpu.VMEM((2, _CHUNK), jnp.float32),
        pltpu.VMEM((_CHUNK,), jnp.float32),
        pltpu.VMEM((4, _CHUNK), jnp.int32),
        pltpu.SemaphoreType.DMA,
        pltpu.SemaphoreType.DMA,
    ],
)(_sc_body)


def kernel(src, flow):
    s = src[..., 0]
    sx = jnp.concatenate([s[:, :, 1:], s[:, :, -1:]], axis=2)
    sy = jnp.concatenate([s[:, 1:, :], s[:, -1:, :]], axis=1)
    sxy = jnp.concatenate([sx[:, 1:, :], sx[:, -1:, :]], axis=1)
    bf = jnp.bfloat16
    ttop = lax.bitcast_convert_type(
        jnp.stack([s.astype(bf), sx.astype(bf)], axis=-1), jnp.int32
    ).reshape(_P)
    tbot = lax.bitcast_convert_type(
        jnp.stack([sy.astype(bf), sxy.astype(bf)], axis=-1), jnp.int32
    ).reshape(_P)
    xs = jnp.arange(_W, dtype=jnp.float32)
    ys = jnp.arange(_H, dtype=jnp.float32)
    gx = (flow[..., 0] + xs[None, None, :]).reshape(_P)
    gy = (flow[..., 1] + ys[None, :, None]).reshape(_P)
    out = _sc_call(ttop, tbot, gx, gy)
    return out.reshape(_B, _H, _W, 1)


# speedup vs baseline: 1.0297x; 1.0297x over previous
import functools

import jax
import jax.numpy as jnp
from jax import lax
from jax.experimental import pallas as pl
from jax.experimental.pallas import tpu as pltpu
from jax.experimental.pallas import tpu_sc as plsc

_B, _H, _W = 8, 512, 512
_P = _B * _H * _W
_NC, _NS, _L = 2, 16, 16
_NW = _NC * _NS
_PIX_PER_W = _P // _NW
_CHUNK = 4096
_NCHUNK = _PIX_PER_W // _CHUNK
_GB = 128
_NGB = _CHUNK // _GB


def _floor_parts(g):
    t = g.astype(jnp.int32)
    tf = t.astype(jnp.float32)
    f = jnp.where(tf > g, tf - 1.0, tf)
    return f.astype(jnp.int32), g - f


def _sc_body(ttop, tbot, gx_hbm, gy_hbm, out_hbm,
             gxv, gyv, idxv, ptv, pbv,
             wav, wbv, wcv, wdv, outv, tmpi, sem0, sem1):
    wid = lax.axis_index("s") * _NC + lax.axis_index("c")

    def compute_and_fire(c, q):
        base = wid * _PIX_PER_W + c * _CHUNK
        bbase = (base >> 18) << 18
        pltpu.sync_copy(gx_hbm.at[pl.ds(base, _CHUNK)], gxv)
        pltpu.sync_copy(gy_hbm.at[pl.ds(base, _CHUNK)], gyv)

        @plsc.parallel_loop(0, _CHUNK // _L, 1, unroll=4)
        def idx_body(i):
            off = i * _L
            gx = gxv[pl.ds(off, _L)]
            gy = gyv[pl.ds(off, _L)]
            x0, fxr = _floor_parts(gx)
            y0, fyr = _floor_parts(gy)
            zero = jnp.zeros_like(fxr)
            fxr = jnp.where(gx < 0.0, zero, fxr)
            fyr = jnp.where(gy < 0.0, zero, fyr)
            exr = 1.0 - fxr
            eyr = 1.0 - fyr
            wa = exr * eyr
            wb = fxr * eyr
            wc = exr * fyr
            wd = fxr * fyr
            x0c = jnp.minimum(jnp.maximum(x0, 0), _W - 1)
            y0c = jnp.minimum(jnp.maximum(y0, 0), _H - 1)
            gidx = bbase + (y0c << 9) + x0c
            idxv[q, pl.ds(off, _L)] = gidx
            wav[q, pl.ds(off, _L)] = wa
            wbv[q, pl.ds(off, _L)] = wb
            wcv[q, pl.ds(off, _L)] = wc
            wdv[q, pl.ds(off, _L)] = wd

        sem = sem0 if q == 0 else sem1

        def fire(j, _):
            sl = pl.ds(j * _GB, _GB)
            isl = idxv.at[q].at[sl]
            pltpu.async_copy(ttop.at[isl], ptv.at[q].at[sl], sem)
            pltpu.async_copy(tbot.at[isl], pbv.at[q].at[sl], sem)
            return 0

        lax.fori_loop(0, _NGB, fire, 0)

    def drain_and_blend(c, q):
        base = wid * _PIX_PER_W + c * _CHUNK
        sem = sem0 if q == 0 else sem1
        dummy = ttop.at[pl.ds(0, _CHUNK)]
        pltpu.make_async_copy(dummy, ptv.at[q], sem).wait()
        pltpu.make_async_copy(dummy, pbv.at[q], sem).wait()
        tmpf = tmpi.bitcast(jnp.float32)

        @plsc.parallel_loop(0, _CHUNK // _L, 1, unroll=4)
        def blend_body(i):
            off = i * _L
            sl = pl.ds(off, _L)
            pt = ptv[q, sl]
            pbt = pbv[q, sl]
            himask = jnp.full_like(pt, -65536)
            tmpi[0, sl] = pt << 16
            tmpi[1, sl] = pt & himask
            tmpi[2, sl] = pbt << 16
            tmpi[3, sl] = pbt & himask
            o = (wav[q, sl] * tmpf[0, sl] + wbv[q, sl] * tmpf[1, sl]
                 + wcv[q, sl] * tmpf[2, sl] + wdv[q, sl] * tmpf[3, sl])
            outv[sl] = o
        pltpu.sync_copy(outv, out_hbm.at[pl.ds(base, _CHUNK)])

    compute_and_fire(0, 0)

    def chunk_pair(m, _):
        c = 2 * m + 1
        compute_and_fire(c, 1)
        drain_and_blend(c - 1, 0)
        compute_and_fire(c + 1, 0)
        drain_and_blend(c, 1)
        return 0

    lax.fori_loop(0, (_NCHUNK - 2) // 2, chunk_pair, 0)
    compute_and_fire(_NCHUNK - 1, 1)
    drain_and_blend(_NCHUNK - 2, 0)
    drain_and_blend(_NCHUNK - 1, 1)


_sc_call = functools.partial(
    pl.kernel,
    out_type=jax.ShapeDtypeStruct((_P,), jnp.float32),
    mesh=plsc.VectorSubcoreMesh(core_axis_name="c", subcore_axis_name="s",
                                num_cores=_NC, num_subcores=_NS),
    scratch_types=[
        pltpu.VMEM((_CHUNK,), jnp.float32),
        pltpu.VMEM((_CHUNK,), jnp.float32),
        pltpu.VMEM((2, _CHUNK), jnp.int32),
        pltpu.VMEM((2, _CHUNK), jnp.int32),
        pltpu.VMEM((2, _CHUNK), jnp.int32),
        pltpu.VMEM((2, _CHUNK), jnp.float32),
        pltpu.VMEM((2, _CHUNK), jnp.float32),
        pltpu.VMEM((2, _CHUNK), jnp.float32),
        pltpu.VMEM((2, _CHUNK), jnp.float32),
        pltpu.VMEM((_CHUNK,), jnp.float32),
        pltpu.VMEM((4, _CHUNK), jnp.int32),
        pltpu.SemaphoreType.DMA,
        pltpu.SemaphoreType.DMA,
    ],
)(_sc_body)


def kernel(src, flow):
    s = src[..., 0]
    sx = jnp.concatenate([s[:, :, 1:], s[:, :, -1:]], axis=2)
    sy = jnp.concatenate([s[:, 1:, :], s[:, -1:, :]], axis=1)
    sxy = jnp.concatenate([sx[:, 1:, :], sx[:, -1:, :]], axis=1)
    bf = jnp.bfloat16
    ttop = lax.bitcast_convert_type(
        jnp.stack([s.astype(bf), sx.astype(bf)], axis=-1), jnp.int32
    ).reshape(_P)
    tbot = lax.bitcast_convert_type(
        jnp.stack([sy.astype(bf), sxy.astype(bf)], axis=-1), jnp.int32
    ).reshape(_P)
    xs = jnp.arange(_W, dtype=jnp.float32)
    ys = jnp.arange(_H, dtype=jnp.float32)
    gx = (flow[..., 0] + xs[None, None, :]).reshape(_P)
    gy = (flow[..., 1] + ys[None, :, None]).reshape(_P)
    out = _sc_call(ttop, tbot, gx, gy)
    return out.reshape(_B, _H, _W, 1)


# speedup vs baseline: 1.0712x; 1.0403x over previous
import functools

import jax
import jax.numpy as jnp
from jax import lax
from jax.experimental import pallas as pl
from jax.experimental.pallas import tpu as pltpu
from jax.experimental.pallas import tpu_sc as plsc

_B, _H, _W = 8, 512, 512
_P = _B * _H * _W
_NC, _NS, _L = 2, 16, 16
_NW = _NC * _NS
_PIX_PER_W = _P // _NW
_CHUNK = 4096
_NCHUNK = _PIX_PER_W // _CHUNK
_GB = 128
_NGB = _CHUNK // _GB


def _sc_body(ttop, tbot, gx_hbm, gy_hbm, out_hbm,
             gxv, gyv, idxv, ptv, pbv,
             wav, wbv, wcv, wdv, outv, tmpi, sem0, sem1):
    wid = lax.axis_index("s") * _NC + lax.axis_index("c")

    def compute_and_fire(c, q):
        base = wid * _PIX_PER_W + c * _CHUNK
        bbase = (base >> 18) << 18
        pltpu.sync_copy(gx_hbm.at[pl.ds(base, _CHUNK)], gxv)
        pltpu.sync_copy(gy_hbm.at[pl.ds(base, _CHUNK)], gyv)

        @plsc.parallel_loop(0, _CHUNK // _L, 1, unroll=4)
        def idx_body(i):
            off = i * _L
            gx = gxv[pl.ds(off, _L)]
            gy = gyv[pl.ds(off, _L)]
            gxc = jnp.minimum(jnp.maximum(gx, 0.0), float(_W - 1))
            gyc = jnp.minimum(jnp.maximum(gy, 0.0), float(_H - 1))
            x0c = gxc.astype(jnp.int32)
            y0c = gyc.astype(jnp.int32)
            fxr = gxc - x0c.astype(jnp.float32)
            fyr = gyc - y0c.astype(jnp.float32)
            exr = 1.0 - fxr
            eyr = 1.0 - fyr
            wa = exr * eyr
            wb = fxr * eyr
            wc = exr * fyr
            wd = fxr * fyr
            gidx = bbase + (y0c << 9) + x0c
            idxv[q, pl.ds(off, _L)] = gidx
            wav[q, pl.ds(off, _L)] = wa
            wbv[q, pl.ds(off, _L)] = wb
            wcv[q, pl.ds(off, _L)] = wc
            wdv[q, pl.ds(off, _L)] = wd

        sem = sem0 if q == 0 else sem1

        def fire(j, _):
            sl = pl.ds(j * _GB, _GB)
            isl = idxv.at[q].at[sl]
            pltpu.async_copy(ttop.at[isl], ptv.at[q].at[sl], sem)
            pltpu.async_copy(tbot.at[isl], pbv.at[q].at[sl], sem)
            return 0

        lax.fori_loop(0, _NGB, fire, 0)

    def drain_and_blend(c, q):
        base = wid * _PIX_PER_W + c * _CHUNK
        sem = sem0 if q == 0 else sem1
        dummy = ttop.at[pl.ds(0, _CHUNK)]
        pltpu.make_async_copy(dummy, ptv.at[q], sem).wait()
        pltpu.make_async_copy(dummy, pbv.at[q], sem).wait()
        tmpf = tmpi.bitcast(jnp.float32)

        @plsc.parallel_loop(0, _CHUNK // _L, 1, unroll=4)
        def blend_body(i):
            off = i * _L
            sl = pl.ds(off, _L)
            pt = ptv[q, sl]
            pbt = pbv[q, sl]
            himask = jnp.full_like(pt, -65536)
            tmpi[0, sl] = pt << 16
            tmpi[1, sl] = pt & himask
            tmpi[2, sl] = pbt << 16
            tmpi[3, sl] = pbt & himask
            o = (wav[q, sl] * tmpf[0, sl] + wbv[q, sl] * tmpf[1, sl]
                 + wcv[q, sl] * tmpf[2, sl] + wdv[q, sl] * tmpf[3, sl])
            outv[sl] = o
        pltpu.sync_copy(outv, out_hbm.at[pl.ds(base, _CHUNK)])

    compute_and_fire(0, 0)

    def chunk_pair(m, _):
        c = 2 * m + 1
        compute_and_fire(c, 1)
        drain_and_blend(c - 1, 0)
        compute_and_fire(c + 1, 0)
        drain_and_blend(c, 1)
        return 0

    lax.fori_loop(0, (_NCHUNK - 2) // 2, chunk_pair, 0)
    compute_and_fire(_NCHUNK - 1, 1)
    drain_and_blend(_NCHUNK - 2, 0)
    drain_and_blend(_NCHUNK - 1, 1)


_sc_call = functools.partial(
    pl.kernel,
    out_type=jax.ShapeDtypeStruct((_P,), jnp.float32),
    mesh=plsc.VectorSubcoreMesh(core_axis_name="c", subcore_axis_name="s",
                                num_cores=_NC, num_subcores=_NS),
    scratch_types=[
        pltpu.VMEM((_CHUNK,), jnp.float32),
        pltpu.VMEM((_CHUNK,), jnp.float32),
        pltpu.VMEM((2, _CHUNK), jnp.int32),
        pltpu.VMEM((2, _CHUNK), jnp.int32),
        pltpu.VMEM((2, _CHUNK), jnp.int32),
        pltpu.VMEM((2, _CHUNK), jnp.float32),
        pltpu.VMEM((2, _CHUNK), jnp.float32),
        pltpu.VMEM((2, _CHUNK), jnp.float32),
        pltpu.VMEM((2, _CHUNK), jnp.float32),
        pltpu.VMEM((_CHUNK,), jnp.float32),
        pltpu.VMEM((4, _CHUNK), jnp.int32),
        pltpu.SemaphoreType.DMA,
        pltpu.SemaphoreType.DMA,
    ],
)(_sc_body)


def kernel(src, flow):
    s = src[..., 0]
    sx = jnp.concatenate([s[:, :, 1:], s[:, :, -1:]], axis=2)
    sy = jnp.concatenate([s[:, 1:, :], s[:, -1:, :]], axis=1)
    sxy = jnp.concatenate([sx[:, 1:, :], sx[:, -1:, :]], axis=1)
    bf = jnp.bfloat16
    ttop = lax.bitcast_convert_type(
        jnp.stack([s.astype(bf), sx.astype(bf)], axis=-1), jnp.int32
    ).reshape(_P)
    tbot = lax.bitcast_convert_type(
        jnp.stack([sy.astype(bf), sxy.astype(bf)], axis=-1), jnp.int32
    ).reshape(_P)
    xs = jnp.arange(_W, dtype=jnp.float32)
    ys = jnp.arange(_H, dtype=jnp.float32)
    gx = (flow[..., 0] + xs[None, None, :]).reshape(_P)
    gy = (flow[..., 1] + ys[None, :, None]).reshape(_P)
    out = _sc_call(ttop, tbot, gx, gy)
    return out.reshape(_B, _H, _W, 1)


# speedup vs baseline: 1.0789x; 1.0071x over previous
import functools

import jax
import jax.numpy as jnp
from jax import lax
from jax.experimental import pallas as pl
from jax.experimental.pallas import tpu as pltpu
from jax.experimental.pallas import tpu_sc as plsc

_B, _H, _W = 8, 512, 512
_P = _B * _H * _W
_NC, _NS, _L = 2, 16, 16
_NW = _NC * _NS
_PIX_PER_W = _P // _NW
_CHUNK = 4096
_NCHUNK = _PIX_PER_W // _CHUNK
_GB = 128
_NGB = _CHUNK // _GB


def _sc_body(ttop, tbot, gx_hbm, gy_hbm, out_hbm,
             gxv, gyv, idxv, ptv, pbv,
             wav, wbv, wcv, wdv, outv, tmpi, sem0, sem1, semg):
    wid = lax.axis_index("s") * _NC + lax.axis_index("c")

    def prefetch(c, q):
        base = wid * _PIX_PER_W + c * _CHUNK
        pltpu.async_copy(gx_hbm.at[pl.ds(base, _CHUNK)], gxv.at[q], semg)
        pltpu.async_copy(gy_hbm.at[pl.ds(base, _CHUNK)], gyv.at[q], semg)

    def compute_and_fire(c, q, prefetch_next):
        base = wid * _PIX_PER_W + c * _CHUNK
        bbase = (base >> 18) << 18
        pltpu.make_async_copy(gx_hbm.at[pl.ds(0, _CHUNK)], gxv.at[q], semg).wait()
        pltpu.make_async_copy(gx_hbm.at[pl.ds(0, _CHUNK)], gyv.at[q], semg).wait()
        if prefetch_next:
            prefetch(c + 1, 1 - q)

        @plsc.parallel_loop(0, _CHUNK // _L, 1, unroll=4)
        def idx_body(i):
            off = i * _L
            gx = gxv[q, pl.ds(off, _L)]
            gy = gyv[q, pl.ds(off, _L)]
            gxc = jnp.minimum(jnp.maximum(gx, 0.0), float(_W - 1))
            gyc = jnp.minimum(jnp.maximum(gy, 0.0), float(_H - 1))
            x0c = gxc.astype(jnp.int32)
            y0c = gyc.astype(jnp.int32)
            fxr = gxc - x0c.astype(jnp.float32)
            fyr = gyc - y0c.astype(jnp.float32)
            exr = 1.0 - fxr
            eyr = 1.0 - fyr
            wa = exr * eyr
            wb = fxr * eyr
            wc = exr * fyr
            wd = fxr * fyr
            gidx = bbase + (y0c << 9) + x0c
            idxv[q, pl.ds(off, _L)] = gidx
            wav[q, pl.ds(off, _L)] = wa
            wbv[q, pl.ds(off, _L)] = wb
            wcv[q, pl.ds(off, _L)] = wc
            wdv[q, pl.ds(off, _L)] = wd

        sem = sem0 if q == 0 else sem1

        def fire(j, _):
            sl = pl.ds(j * _GB, _GB)
            isl = idxv.at[q].at[sl]
            pltpu.async_copy(ttop.at[isl], ptv.at[q].at[sl], sem)
            pltpu.async_copy(tbot.at[isl], pbv.at[q].at[sl], sem)
            return 0

        lax.fori_loop(0, _NGB, fire, 0)

    def drain_and_blend(c, q):
        base = wid * _PIX_PER_W + c * _CHUNK
        sem = sem0 if q == 0 else sem1
        dummy = ttop.at[pl.ds(0, _CHUNK)]
        pltpu.make_async_copy(dummy, ptv.at[q], sem).wait()
        pltpu.make_async_copy(dummy, pbv.at[q], sem).wait()
        tmpf = tmpi.bitcast(jnp.float32)
        ptf = ptv.bitcast(jnp.float32)
        pbf = pbv.bitcast(jnp.float32)

        @plsc.parallel_loop(0, _CHUNK // _L, 1, unroll=4)
        def blend_body(i):
            off = i * _L
            sl = pl.ds(off, _L)
            tmpi[0, sl] = ptv[q, sl] << 16
            tmpi[1, sl] = pbv[q, sl] << 16
            o = (wav[q, sl] * tmpf[0, sl] + wbv[q, sl] * ptf[q, sl]
                 + wcv[q, sl] * tmpf[1, sl] + wdv[q, sl] * pbf[q, sl])
            outv[sl] = o
        pltpu.sync_copy(outv, out_hbm.at[pl.ds(base, _CHUNK)])

    prefetch(0, 0)
    compute_and_fire(0, 0, True)

    def chunk_pair(m, _):
        c = 2 * m + 1
        compute_and_fire(c, 1, True)
        drain_and_blend(c - 1, 0)
        compute_and_fire(c + 1, 0, True)
        drain_and_blend(c, 1)
        return 0

    lax.fori_loop(0, (_NCHUNK - 2) // 2, chunk_pair, 0)
    compute_and_fire(_NCHUNK - 1, 1, False)
    drain_and_blend(_NCHUNK - 2, 0)
    drain_and_blend(_NCHUNK - 1, 1)


_sc_call = functools.partial(
    pl.kernel,
    out_type=jax.ShapeDtypeStruct((_P,), jnp.float32),
    mesh=plsc.VectorSubcoreMesh(core_axis_name="c", subcore_axis_name="s",
                                num_cores=_NC, num_subcores=_NS),
    scratch_types=[
        pltpu.VMEM((2, _CHUNK), jnp.float32),
        pltpu.VMEM((2, _CHUNK), jnp.float32),
        pltpu.VMEM((2, _CHUNK), jnp.int32),
        pltpu.VMEM((2, _CHUNK), jnp.int32),
        pltpu.VMEM((2, _CHUNK), jnp.int32),
        pltpu.VMEM((2, _CHUNK), jnp.float32),
        pltpu.VMEM((2, _CHUNK), jnp.float32),
        pltpu.VMEM((2, _CHUNK), jnp.float32),
        pltpu.VMEM((2, _CHUNK), jnp.float32),
        pltpu.VMEM((_CHUNK,), jnp.float32),
        pltpu.VMEM((2, _CHUNK), jnp.int32),
        pltpu.SemaphoreType.DMA,
        pltpu.SemaphoreType.DMA,
        pltpu.SemaphoreType.DMA,
    ],
)(_sc_body)


def kernel(src, flow):
    s = src[..., 0]
    sx = jnp.concatenate([s[:, :, 1:], s[:, :, -1:]], axis=2)
    sy = jnp.concatenate([s[:, 1:, :], s[:, -1:, :]], axis=1)
    sxy = jnp.concatenate([sx[:, 1:, :], sx[:, -1:, :]], axis=1)
    bf = jnp.bfloat16
    ttop = lax.bitcast_convert_type(
        jnp.stack([s.astype(bf), sx.astype(bf)], axis=-1), jnp.int32
    ).reshape(_P)
    tbot = lax.bitcast_convert_type(
        jnp.stack([sy.astype(bf), sxy.astype(bf)], axis=-1), jnp.int32
    ).reshape(_P)
    xs = jnp.arange(_W, dtype=jnp.float32)
    ys = jnp.arange(_H, dtype=jnp.float32)
    gx = (flow[..., 0] + xs[None, None, :]).reshape(_P)
    gy = (flow[..., 1] + ys[None, :, None]).reshape(_P)
    out = _sc_call(ttop, tbot, gx, gy)
    return out.reshape(_B, _H, _W, 1)


# speedup vs baseline: 1.0896x; 1.0099x over previous
import functools

import jax
import jax.numpy as jnp
from jax import lax
from jax.experimental import pallas as pl
from jax.experimental.pallas import tpu as pltpu
from jax.experimental.pallas import tpu_sc as plsc

_B, _H, _W = 8, 512, 512
_P = _B * _H * _W
_NC, _NS, _L = 2, 16, 16
_NW = _NC * _NS
_PIX_PER_W = _P // _NW
_CHUNK = 4096
_NCHUNK = _PIX_PER_W // _CHUNK
_GB = 128
_NGB = _CHUNK // _GB


def _sc_body(ttop, tbot, gx_hbm, gy_hbm, out_hbm,
             gxv, gyv, idxv, ptv, pbv,
             wav, wbv, wcv, wdv, outv, tmpi, sem0, sem1, semg):
    wid = lax.axis_index("s") * _NC + lax.axis_index("c")

    def prefetch(c, q):
        base = wid * _PIX_PER_W + c * _CHUNK
        pltpu.async_copy(gx_hbm.at[pl.ds(base, _CHUNK)], gxv.at[q], semg)
        pltpu.async_copy(gy_hbm.at[pl.ds(base, _CHUNK)], gyv.at[q], semg)

    def compute_and_fire(c, q, prefetch_next):
        base = wid * _PIX_PER_W + c * _CHUNK
        bbase = (base >> 18) << 18
        pltpu.make_async_copy(gx_hbm.at[pl.ds(0, _CHUNK)], gxv.at[q], semg).wait()
        pltpu.make_async_copy(gx_hbm.at[pl.ds(0, _CHUNK)], gyv.at[q], semg).wait()
        if prefetch_next:
            prefetch(c + 1, 1 - q)

        sem = sem0 if q == 0 else sem1

        def batch_body(j, _):
            @plsc.parallel_loop(j * (_GB // _L), (j + 1) * (_GB // _L), 1,
                                unroll=4)
            def idx_body(i):
                off = i * _L
                gx = gxv[q, pl.ds(off, _L)]
                gy = gyv[q, pl.ds(off, _L)]
                gxc = jnp.minimum(jnp.maximum(gx, 0.0), float(_W - 1))
                gyc = jnp.minimum(jnp.maximum(gy, 0.0), float(_H - 1))
                x0c = gxc.astype(jnp.int32)
                y0c = gyc.astype(jnp.int32)
                fxr = gxc - x0c.astype(jnp.float32)
                fyr = gyc - y0c.astype(jnp.float32)
                exr = 1.0 - fxr
                eyr = 1.0 - fyr
                wa = exr * eyr
                wb = fxr * eyr
                wc = exr * fyr
                wd = fxr * fyr
                gidx = bbase + (y0c << 9) + x0c
                idxv[q, pl.ds(off, _L)] = gidx
                wav[q, pl.ds(off, _L)] = wa
                wbv[q, pl.ds(off, _L)] = wb
                wcv[q, pl.ds(off, _L)] = wc
                wdv[q, pl.ds(off, _L)] = wd

            sl = pl.ds(j * _GB, _GB)
            isl = idxv.at[q].at[sl]
            pltpu.async_copy(ttop.at[isl], ptv.at[q].at[sl], sem)
            pltpu.async_copy(tbot.at[isl], pbv.at[q].at[sl], sem)
            return 0

        lax.fori_loop(0, _NGB, batch_body, 0)

    def drain_and_blend(c, q):
        base = wid * _PIX_PER_W + c * _CHUNK
        sem = sem0 if q == 0 else sem1
        dummy = ttop.at[pl.ds(0, _CHUNK)]
        pltpu.make_async_copy(dummy, ptv.at[q], sem).wait()
        pltpu.make_async_copy(dummy, pbv.at[q], sem).wait()
        tmpf = tmpi.bitcast(jnp.float32)
        ptf = ptv.bitcast(jnp.float32)
        pbf = pbv.bitcast(jnp.float32)

        @plsc.parallel_loop(0, _CHUNK // _L, 1, unroll=4)
        def blend_body(i):
            off = i * _L
            sl = pl.ds(off, _L)
            tmpi[0, sl] = ptv[q, sl] << 16
            tmpi[1, sl] = pbv[q, sl] << 16
            o = (wav[q, sl] * tmpf[0, sl] + wbv[q, sl] * ptf[q, sl]
                 + wcv[q, sl] * tmpf[1, sl] + wdv[q, sl] * pbf[q, sl])
            outv[sl] = o
        pltpu.sync_copy(outv, out_hbm.at[pl.ds(base, _CHUNK)])

    prefetch(0, 0)
    compute_and_fire(0, 0, True)

    def chunk_pair(m, _):
        c = 2 * m + 1
        compute_and_fire(c, 1, True)
        drain_and_blend(c - 1, 0)
        compute_and_fire(c + 1, 0, True)
        drain_and_blend(c, 1)
        return 0

    lax.fori_loop(0, (_NCHUNK - 2) // 2, chunk_pair, 0)
    compute_and_fire(_NCHUNK - 1, 1, False)
    drain_and_blend(_NCHUNK - 2, 0)
    drain_and_blend(_NCHUNK - 1, 1)


_sc_call = functools.partial(
    pl.kernel,
    out_type=jax.ShapeDtypeStruct((_P,), jnp.float32),
    mesh=plsc.VectorSubcoreMesh(core_axis_name="c", subcore_axis_name="s",
                                num_cores=_NC, num_subcores=_NS),
    scratch_types=[
        pltpu.VMEM((2, _CHUNK), jnp.float32),
        pltpu.VMEM((2, _CHUNK), jnp.float32),
        pltpu.VMEM((2, _CHUNK), jnp.int32),
        pltpu.VMEM((2, _CHUNK), jnp.int32),
        pltpu.VMEM((2, _CHUNK), jnp.int32),
        pltpu.VMEM((2, _CHUNK), jnp.float32),
        pltpu.VMEM((2, _CHUNK), jnp.float32),
        pltpu.VMEM((2, _CHUNK), jnp.float32),
        pltpu.VMEM((2, _CHUNK), jnp.float32),
        pltpu.VMEM((_CHUNK,), jnp.float32),
        pltpu.VMEM((2, _CHUNK), jnp.int32),
        pltpu.SemaphoreType.DMA,
        pltpu.SemaphoreType.DMA,
        pltpu.SemaphoreType.DMA,
    ],
)(_sc_body)


def kernel(src, flow):
    s = src[..., 0]
    sx = jnp.concatenate([s[:, :, 1:], s[:, :, -1:]], axis=2)
    sy = jnp.concatenate([s[:, 1:, :], s[:, -1:, :]], axis=1)
    sxy = jnp.concatenate([sx[:, 1:, :], sx[:, -1:, :]], axis=1)
    bf = jnp.bfloat16
    ttop = lax.bitcast_convert_type(
        jnp.stack([s.astype(bf), sx.astype(bf)], axis=-1), jnp.int32
    ).reshape(_P)
    tbot = lax.bitcast_convert_type(
        jnp.stack([sy.astype(bf), sxy.astype(bf)], axis=-1), jnp.int32
    ).reshape(_P)
    xs = jnp.arange(_W, dtype=jnp.float32)
    ys = jnp.arange(_H, dtype=jnp.float32)
    gx = (flow[..., 0] + xs[None, None, :]).reshape(_P)
    gy = (flow[..., 1] + ys[None, :, None]).reshape(_P)
    out = _sc_call(ttop, tbot, gx, gy)
    return out.reshape(_B, _H, _W, 1)
